# Initial kernel scaffold; baseline (speedup 1.0000x reference)
import functools

import jax
import jax.numpy as jnp
from jax import lax
from jax.experimental import pallas as pl
from jax.experimental.pallas import tpu as pltpu
from jax.experimental.pallas import tpu_sc as plsc

DIM = 64
CHUNK = 128


def kernel(xinput, weights):
    B, L = xinput.shape
    info = plsc.get_sparse_core_info()
    nw = info.num_cores * info.num_subcores
    nchunks = B // CHUNK
    chunks_per_w = nchunks // nw

    idx3 = xinput.reshape(nchunks, CHUNK, L).transpose(0, 2, 1)

    @functools.partial(
        pl.kernel,
        mesh=plsc.VectorSubcoreMesh(core_axis_name="c", subcore_axis_name="s"),
        out_type=jax.ShapeDtypeStruct((B, DIM), jnp.float32),
        scratch_types=[
            pltpu.VMEM((L, CHUNK), jnp.int32),
            pltpu.VMEM((CHUNK, DIM), jnp.float32),
            pltpu.VMEM((CHUNK, DIM), jnp.float32),
            pltpu.SemaphoreType.DMA,
        ],
    )
    def sc_kernel(idx_hbm, table_hbm, out_hbm, idx_v, acc, buf, sem):
        wid = lax.axis_index("s") * info.num_cores + lax.axis_index("c")

        def do_chunk(c, _):
            cb = wid * chunks_per_w + c
            pltpu.sync_copy(idx_hbm.at[cb], idx_v)
            pltpu.async_copy(table_hbm.at[idx_v.at[0]], acc, sem).wait()

            def do_l(l, _):
                pltpu.async_copy(table_hbm.at[idx_v.at[l]], buf, sem).wait()

                def do_row(j, _):
                    for d in range(DIM // 16):
                        sl = pl.ds(16 * d, 16)
                        plsc.addupdate(acc.at[j, sl], buf[j, sl])
                    return 0

                lax.fori_loop(0, CHUNK, do_row, 0)
                return 0

            lax.fori_loop(1, L, do_l, 0)
            pltpu.sync_copy(acc, out_hbm.at[pl.ds(cb * CHUNK, CHUNK)])
            return 0

        lax.fori_loop(0, chunks_per_w, do_chunk, 0)

    return sc_kernel(idx3, weights)


# speedup vs baseline: 2.1611x; 2.1611x over previous
import functools

import jax
import jax.numpy as jnp
from jax import lax
from jax.experimental import pallas as pl
from jax.experimental.pallas import tpu as pltpu
from jax.experimental.pallas import tpu_sc as plsc

DIM = 64
CHUNK = 128


def kernel(xinput, weights):
    B, L = xinput.shape
    info = plsc.get_sparse_core_info()
    nw = info.num_cores * info.num_subcores
    nchunks = B // CHUNK
    chunks_per_w = nchunks // nw

    idx3 = xinput.reshape(nchunks, CHUNK, L).transpose(0, 2, 1)

    @functools.partial(
        pl.kernel,
        mesh=plsc.VectorSubcoreMesh(core_axis_name="c", subcore_axis_name="s"),
        out_type=jax.ShapeDtypeStruct((B, DIM), jnp.float32),
        scratch_types=[
            pltpu.VMEM((L, CHUNK), jnp.int32),
            pltpu.VMEM((CHUNK, DIM), jnp.float32),
            pltpu.VMEM((CHUNK, DIM), jnp.float32),
            pltpu.SemaphoreType.DMA,
        ],
        compiler_params=pltpu.CompilerParams(use_tc_tiling_on_sc=False),
    )
    def sc_kernel(idx_hbm, table_hbm, out_hbm, idx_v, acc, buf, sem):
        wid = lax.axis_index("s") * info.num_cores + lax.axis_index("c")

        def do_chunk(c, _):
            cb = wid * chunks_per_w + c
            pltpu.sync_copy(idx_hbm.at[cb], idx_v)
            pltpu.async_copy(table_hbm.at[idx_v.at[0]], acc, sem).wait()

            def do_l(l, _):
                pltpu.async_copy(table_hbm.at[idx_v.at[l]], buf, sem).wait()

                def do_row(j, _):
                    for d in range(DIM // 16):
                        sl = pl.ds(16 * d, 16)
                        plsc.addupdate(acc.at[j, sl], buf[j, sl])
                    return 0

                lax.fori_loop(0, CHUNK, do_row, 0)
                return 0

            lax.fori_loop(1, L, do_l, 0)
            pltpu.sync_copy(acc, out_hbm.at[pl.ds(cb * CHUNK, CHUNK)])
            return 0

        lax.fori_loop(0, chunks_per_w, do_chunk, 0)

    return sc_kernel(idx3, weights)


# speedup vs baseline: 2.8753x; 1.3304x over previous
import functools

import jax
import jax.numpy as jnp
from jax import lax
from jax.experimental import pallas as pl
from jax.experimental.pallas import tpu as pltpu
from jax.experimental.pallas import tpu_sc as plsc

DIM = 64
CHUNK = 128


def kernel(xinput, weights):
    B, L = xinput.shape
    info = plsc.get_sparse_core_info()
    nw = info.num_cores * info.num_subcores
    nchunks = B // CHUNK
    cpw = nchunks // nw

    idx3 = xinput.reshape(nchunks, CHUNK, L).transpose(0, 2, 1)

    @functools.partial(
        pl.kernel,
        mesh=plsc.VectorSubcoreMesh(core_axis_name="c", subcore_axis_name="s"),
        out_type=jax.ShapeDtypeStruct((B, DIM), jnp.float32),
        scratch_types=[
            pltpu.VMEM((cpw, L, CHUNK), jnp.int32),
            pltpu.VMEM((cpw, CHUNK, DIM), jnp.float32),
        ]
        + [pltpu.SemaphoreType.DMA] * cpw,
        compiler_params=pltpu.CompilerParams(use_tc_tiling_on_sc=False),
    )
    def sc_kernel(idx_hbm, table_hbm, out_hbm, idx_v, acc, *sems):
        wid = lax.axis_index("s") * info.num_cores + lax.axis_index("c")

        for c in range(cpw):
            pltpu.sync_copy(idx_hbm.at[wid * cpw + c], idx_v.at[c])

        zero = jnp.zeros((16,), jnp.float32)

        def zero_row(j, _):
            for c in range(cpw):
                for d in range(DIM // 16):
                    acc[c, j, pl.ds(16 * d, 16)] = zero
            return 0

        lax.fori_loop(0, CHUNK, zero_row, 0)

        for c in range(cpw):

            def fire(l, _, c=c):
                pltpu.async_copy(
                    table_hbm.at[idx_v.at[c, l]], acc.at[c], sems[c], add=True
                )
                return 0

            lax.fori_loop(0, L, fire, 0)

        for c in range(cpw):

            def drain(l, _, c=c):
                pltpu.make_async_copy(
                    table_hbm.at[idx_v.at[c, 0]], acc.at[c], sems[c]
                ).wait()
                return 0

            lax.fori_loop(0, L, drain, 0)
            pltpu.sync_copy(acc.at[c], out_hbm.at[pl.ds((wid * cpw + c) * CHUNK, CHUNK)])

    return sc_kernel(idx3, weights)
